# Initial kernel scaffold; baseline (speedup 1.0000x reference)
import jax
import jax.numpy as jnp
from jax.experimental import pallas as pl


def kernel(fragments_idx, features_packed):
    raise NotImplementedError("write your pallas kernel here")


# speedup vs baseline: 32.2121x; 32.2121x over previous
import functools

import jax
import jax.numpy as jnp
from jax import lax
from jax.experimental import pallas as pl
from jax.experimental.pallas import tpu as pltpu
from jax.experimental.pallas import tpu_sc as plsc

B, H, W, K = 4, 512, 512, 8
P, C = 1000000, 3
N = B * H * W
D = 8

_info = plsc.get_sparse_core_info()
NC, NS, L = _info.num_cores, _info.num_subcores, _info.num_lanes
NW = NC * NS
PER_W = N // NW
S = 8192
NCHUNK = PER_W // S

_mesh = plsc.VectorSubcoreMesh(core_axis_name="c", subcore_axis_name="s")


@functools.partial(
    pl.kernel,
    mesh=_mesh,
    out_type=jax.ShapeDtypeStruct((N, D), jnp.float32),
    scratch_types=[
        pltpu.VMEM((S,), jnp.int32),
        pltpu.VMEM((S,), jnp.int32),
        pltpu.VMEM((S, D), jnp.float32),
        pltpu.SemaphoreType.DMA,
    ],
    compiler_params=pltpu.CompilerParams(use_tc_tiling_on_sc=False),
)
def _render(idx_hbm, table_hbm, out_hbm, idx_v, sidx_v, rows_v, sem):
    wid = lax.axis_index("s") * NC + lax.axis_index("c")
    base = wid * PER_W
    bg_row = jnp.full((L,), P, jnp.int32)

    for ch in range(NCHUNK):
        off = base + ch * S
        pltpu.sync_copy(idx_hbm.at[pl.ds(off, S)], idx_v)

        def remap_body(i, _):
            v = idx_v[pl.ds(i * L, L)]
            sidx_v[pl.ds(i * L, L)] = jnp.where(v < 0, bg_row, v)
            return 0

        lax.fori_loop(0, S // L, remap_body, 0)

        pltpu.async_copy(table_hbm.at[sidx_v], rows_v, sem).wait()

        pltpu.sync_copy(rows_v, out_hbm.at[pl.ds(off, S)])


def kernel(fragments_idx, features_packed):
    idx0 = fragments_idx[..., 0].reshape(N)
    table = jnp.pad(
        jnp.concatenate(
            [features_packed, jnp.ones((8, C), jnp.float32)], axis=0),
        ((0, 0), (0, D - C)))
    out = _render(idx0, table)
    return out[:, :C].reshape(B, H, W, C)
